# Initial kernel scaffold; baseline (speedup 1.0000x reference)
import jax
import jax.numpy as jnp
from jax.experimental import pallas as pl


def kernel(queries, keys, values):
    raise NotImplementedError("write your pallas kernel here")


# speedup vs baseline: 13.5846x; 13.5846x over previous
import functools

import jax
import jax.numpy as jnp
from jax.experimental import pallas as pl
from jax.experimental.pallas import tpu as pltpu

EPS = 1e-06
BQ = 256
BK = 512


def _tri(d):
    return (jax.lax.broadcasted_iota(jnp.int32, (d, d), 0)
            <= jax.lax.broadcasted_iota(jnp.int32, (d, d), 1)
            ).astype(jnp.float32)


def _onehot_ranked(x, tri):
    ax = jnp.abs(x)
    mx = jnp.max(ax, axis=-1, keepdims=True)
    eq = (ax == mx).astype(jnp.float32)
    cnt = jax.lax.dot_general(
        eq, tri, (((1,), (0,)), ((), ())),
        preferred_element_type=jnp.float32,
        precision=jax.lax.Precision.HIGHEST,
    )
    h1 = eq * (cnt == 1.0).astype(jnp.float32)
    h2 = eq * (cnt == 2.0).astype(jnp.float32)
    return h1, h2


def _assign_body(k_ref, a_ref):
    k = k_ref[0]
    d = k.shape[-1]
    h1, _ = _onehot_ranked(k, _tri(d))
    iota_col = jax.lax.broadcasted_iota(jnp.int32, (d, 1), 0).astype(jnp.float32)
    iota_row = iota_col.reshape(1, d)
    a = jax.lax.dot_general(
        iota_row, h1, (((1,), (1,)), ((), ())),
        preferred_element_type=jnp.float32,
        precision=jax.lax.Precision.HIGHEST,
    )
    a_ref[0] = a.astype(jnp.int32)


def _z_body(a_ref, k_ref, z_ref):
    s = k_ref.shape[1]
    z_ref[...] = jnp.zeros_like(z_ref)

    def step(i, _):
        c = a_ref[0, 0, i]
        row = k_ref[0, pl.ds(i, 1), :]
        z_ref[0, pl.ds(c, 1), :] += row
        return 0

    jax.lax.fori_loop(0, s, step, 0)


def _attn_body(q_ref, k_ref, v_ref, z_ref, o_ref):
    q = q_ref[0]
    s = k_ref.shape[1]
    d = q.shape[-1]

    tri = _tri(d)
    hq1, hq2 = _onehot_ranked(q, tri)

    z = z_ref[0]

    def den_of(hq):
        zsel = jax.lax.dot_general(
            hq, z, (((1,), (0,)), ((), ())),
            preferred_element_type=jnp.float32,
            precision=jax.lax.Precision.HIGHEST,
        )
        p = zsel * q
        t = p + jnp.roll(p, -1, axis=1)
        t = t + jnp.roll(p, -2, axis=1)
        t = t + jnp.roll(p, -3, axis=1)
        t = t + jnp.roll(t, -4, axis=1)
        t = t + jnp.roll(t, -32, axis=1)
        t = t + jnp.roll(t, -16, axis=1)
        t = t + jnp.roll(t, -8, axis=1)
        return t[:, 0:1]

    den1 = den_of(hq1)
    den2 = den_of(hq2)

    def body(jb, carry):
        num1, num2 = carry
        kb = k_ref[0, pl.ds(jb * BK, BK), :]
        vb = v_ref[0, pl.ds(jb * BK, BK), :]

        hk, _ = _onehot_ranked(kb, tri)

        A = jax.lax.dot_general(
            q, kb, (((1,), (1,)), ((), ())),
            preferred_element_type=jnp.float32,
            precision=jax.lax.Precision.HIGHEST,
        )

        def term(hq, num):
            mask = jax.lax.dot_general(
                hq, hk, (((1,), (1,)), ((), ())),
                preferred_element_type=jnp.float32,
                precision=jax.lax.Precision.DEFAULT,
            )
            Am = A * mask
            return num + jax.lax.dot_general(
                Am, vb, (((1,), (0,)), ((), ())),
                preferred_element_type=jnp.float32,
                precision=jax.lax.Precision.HIGHEST,
            )

        return term(hq1, num1), term(hq2, num2)

    zeros_nd = jnp.zeros((BQ, d), jnp.float32)
    num1, num2 = jax.lax.fori_loop(0, s // BK, body, (zeros_nd, zeros_nd))

    o_ref[0] = 0.125 * (num1 / (den1 + EPS) + num2 / (den2 + EPS))


def kernel(queries, keys, values):
    b, h, s, d = queries.shape
    q = queries.reshape(b * h, s, d)
    k = keys.reshape(b * h, s, d)
    v = values.reshape(b * h, s, d)
    nh = b * h

    assigns = pl.pallas_call(
        _assign_body,
        grid=(nh,),
        in_specs=[pl.BlockSpec((1, s, d), lambda hh: (hh, 0, 0))],
        out_specs=pl.BlockSpec((1, 1, s), lambda hh: (hh, 0, 0)),
        out_shape=jax.ShapeDtypeStruct((nh, 1, s), jnp.int32),
    )(k)

    zmat = pl.pallas_call(
        _z_body,
        grid=(nh,),
        in_specs=[
            pl.BlockSpec((1, 1, s), lambda hh: (hh, 0, 0),
                         memory_space=pltpu.SMEM),
            pl.BlockSpec((1, s, d), lambda hh: (hh, 0, 0)),
        ],
        out_specs=pl.BlockSpec((1, d, d), lambda hh: (hh, 0, 0)),
        out_shape=jax.ShapeDtypeStruct((nh, d, d), jnp.float32),
    )(assigns, k)

    grid = (nh, s // BQ)
    out = pl.pallas_call(
        _attn_body,
        grid=grid,
        in_specs=[
            pl.BlockSpec((1, BQ, d), lambda hh, i: (hh, i, 0)),
            pl.BlockSpec((1, s, d), lambda hh, i: (hh, 0, 0)),
            pl.BlockSpec((1, s, d), lambda hh, i: (hh, 0, 0)),
            pl.BlockSpec((1, d, d), lambda hh, i: (hh, 0, 0)),
        ],
        out_specs=pl.BlockSpec((1, BQ, d), lambda hh, i: (hh, i, 0)),
        out_shape=jax.ShapeDtypeStruct((nh, s, d), jnp.float32),
    )(q, k, v, zmat)
    return out.reshape(b, h, s, d)


# speedup vs baseline: 14.1514x; 1.0417x over previous
import functools

import jax
import jax.numpy as jnp
from jax.experimental import pallas as pl
from jax.experimental.pallas import tpu as pltpu

EPS = 1e-06
BQ = 256
BK = 512


def _tri(d):
    return (jax.lax.broadcasted_iota(jnp.int32, (d, d), 0)
            <= jax.lax.broadcasted_iota(jnp.int32, (d, d), 1)
            ).astype(jnp.float32)


def _onehot_ranked(x, tri):
    ax = jnp.abs(x)
    mx = jnp.max(ax, axis=-1, keepdims=True)
    eq = (ax == mx).astype(jnp.float32)
    cnt = jax.lax.dot_general(
        eq, tri, (((1,), (0,)), ((), ())),
        preferred_element_type=jnp.float32,
        precision=jax.lax.Precision.HIGHEST,
    )
    h1 = eq * (cnt == 1.0).astype(jnp.float32)
    h2 = eq * (cnt == 2.0).astype(jnp.float32)
    return h1, h2


def _assign_body(k_ref, a_ref):
    k = k_ref[0]
    d = k.shape[-1]
    h1, _ = _onehot_ranked(k, _tri(d))
    iota_col = jax.lax.broadcasted_iota(jnp.int32, (d, 1), 0).astype(jnp.float32)
    iota_row = iota_col.reshape(1, d)
    a = jax.lax.dot_general(
        iota_row, h1, (((1,), (1,)), ((), ())),
        preferred_element_type=jnp.float32,
        precision=jax.lax.Precision.HIGHEST,
    )
    a_ref[0] = a.astype(jnp.int32)


def _z_body(a_ref, k_ref, z_ref):
    s = k_ref.shape[1]
    z_ref[...] = jnp.zeros_like(z_ref)

    def step(i, _):
        c = a_ref[0, 0, i]
        row = k_ref[0, pl.ds(i, 1), :]
        z_ref[0, pl.ds(c, 1), :] += row
        return 0

    jax.lax.fori_loop(0, s, step, 0)


def _attn_body(q_ref, k_ref, v_ref, z_ref, o_ref):
    q = q_ref[0]
    s = k_ref.shape[1]
    d = q.shape[-1]

    tri = _tri(d)
    hq1, hq2 = _onehot_ranked(q, tri)

    z = z_ref[0]

    def den_of(hq):
        zsel = jax.lax.dot_general(
            hq, z, (((1,), (0,)), ((), ())),
            preferred_element_type=jnp.float32,
            precision=jax.lax.Precision.HIGHEST,
        )
        p = zsel * q
        t = p + jnp.roll(p, -1, axis=1)
        t = t + jnp.roll(p, -2, axis=1)
        t = t + jnp.roll(p, -3, axis=1)
        t = t + jnp.roll(t, -4, axis=1)
        t = t + jnp.roll(t, -32, axis=1)
        t = t + jnp.roll(t, -16, axis=1)
        t = t + jnp.roll(t, -8, axis=1)
        return t[:, 0:1]

    den1 = den_of(hq1)
    den2 = den_of(hq2)

    w = 0.125 * (hq1 / (den1 + EPS) + hq2 / (den2 + EPS))

    def body(jb, num):
        kb = k_ref[0, pl.ds(jb * BK, BK), :]
        vb = v_ref[0, pl.ds(jb * BK, BK), :]

        hk, _ = _onehot_ranked(kb, tri)

        A = jax.lax.dot_general(
            q, kb, (((1,), (1,)), ((), ())),
            preferred_element_type=jnp.float32,
            precision=jax.lax.Precision.HIGHEST,
        )
        G = jax.lax.dot_general(
            w, hk, (((1,), (1,)), ((), ())),
            preferred_element_type=jnp.float32,
            precision=jax.lax.Precision.HIGHEST,
        )
        return num + jax.lax.dot_general(
            A * G, vb, (((1,), (0,)), ((), ())),
            preferred_element_type=jnp.float32,
            precision=jax.lax.Precision.HIGHEST,
        )

    num = jax.lax.fori_loop(0, s // BK, body, jnp.zeros((BQ, d), jnp.float32))
    o_ref[0] = num


def kernel(queries, keys, values):
    b, h, s, d = queries.shape
    q = queries.reshape(b * h, s, d)
    k = keys.reshape(b * h, s, d)
    v = values.reshape(b * h, s, d)
    nh = b * h

    assigns = pl.pallas_call(
        _assign_body,
        grid=(nh,),
        in_specs=[pl.BlockSpec((1, s, d), lambda hh: (hh, 0, 0))],
        out_specs=pl.BlockSpec((1, 1, s), lambda hh: (hh, 0, 0)),
        out_shape=jax.ShapeDtypeStruct((nh, 1, s), jnp.int32),
    )(k)

    zmat = pl.pallas_call(
        _z_body,
        grid=(nh,),
        in_specs=[
            pl.BlockSpec((1, 1, s), lambda hh: (hh, 0, 0),
                         memory_space=pltpu.SMEM),
            pl.BlockSpec((1, s, d), lambda hh: (hh, 0, 0)),
        ],
        out_specs=pl.BlockSpec((1, d, d), lambda hh: (hh, 0, 0)),
        out_shape=jax.ShapeDtypeStruct((nh, d, d), jnp.float32),
    )(assigns, k)

    grid = (nh, s // BQ)
    out = pl.pallas_call(
        _attn_body,
        grid=grid,
        in_specs=[
            pl.BlockSpec((1, BQ, d), lambda hh, i: (hh, i, 0)),
            pl.BlockSpec((1, s, d), lambda hh, i: (hh, 0, 0)),
            pl.BlockSpec((1, s, d), lambda hh, i: (hh, 0, 0)),
            pl.BlockSpec((1, d, d), lambda hh, i: (hh, 0, 0)),
        ],
        out_specs=pl.BlockSpec((1, BQ, d), lambda hh, i: (hh, i, 0)),
        out_shape=jax.ShapeDtypeStruct((nh, s, d), jnp.float32),
    )(q, k, v, zmat)
    return out.reshape(b, h, s, d)


# speedup vs baseline: 15.7408x; 1.1123x over previous
import functools

import jax
import jax.numpy as jnp
from jax.experimental import pallas as pl
from jax.experimental.pallas import tpu as pltpu

EPS = 1e-06
BQ = 256
BK = 512


def _tri(d):
    return (jax.lax.broadcasted_iota(jnp.int32, (d, d), 0)
            <= jax.lax.broadcasted_iota(jnp.int32, (d, d), 1)
            ).astype(jnp.float32)


def _onehot_ranked(x, tri):
    ax = jnp.abs(x)
    mx = jnp.max(ax, axis=-1, keepdims=True)
    eq = (ax == mx).astype(jnp.float32)
    cnt = jax.lax.dot_general(
        eq, tri, (((1,), (0,)), ((), ())),
        preferred_element_type=jnp.float32,
        precision=jax.lax.Precision.HIGHEST,
    )
    h1 = eq * (cnt == 1.0).astype(jnp.float32)
    h2 = eq * (cnt == 2.0).astype(jnp.float32)
    return h1, h2


def _assign_body(k_ref, a_ref):
    k = k_ref[0]
    d = k.shape[-1]
    h1, _ = _onehot_ranked(k, _tri(d))
    iota_col = jax.lax.broadcasted_iota(jnp.int32, (d, 1), 0).astype(jnp.float32)
    iota_row = iota_col.reshape(1, d)
    a = jax.lax.dot_general(
        iota_row, h1, (((1,), (1,)), ((), ())),
        preferred_element_type=jnp.float32,
        precision=jax.lax.Precision.HIGHEST,
    )
    a_ref[0] = a.astype(jnp.int32)


def _z_body(a_ref, k_ref, z_ref):
    nh = k_ref.shape[0]
    s = k_ref.shape[1]
    z_ref[...] = jnp.zeros_like(z_ref)

    def step(i, _):
        for h in range(nh):
            c = a_ref[h, 0, i]
            z_ref[h, pl.ds(c, 1), :] += k_ref[h, pl.ds(i, 1), :]
        return 0

    jax.lax.fori_loop(0, s, step, 0)


def _attn_body(q_ref, k_ref, v_ref, z_ref, o_ref):
    q = q_ref[0]
    s = k_ref.shape[1]
    d = q.shape[-1]

    tri = _tri(d)
    hq1, hq2 = _onehot_ranked(q, tri)

    z = z_ref[0]

    def den_of(hq):
        zsel = jax.lax.dot_general(
            hq, z, (((1,), (0,)), ((), ())),
            preferred_element_type=jnp.float32,
            precision=jax.lax.Precision.HIGHEST,
        )
        p = zsel * q
        t = p + jnp.roll(p, -1, axis=1)
        t = t + jnp.roll(p, -2, axis=1)
        t = t + jnp.roll(p, -3, axis=1)
        t = t + jnp.roll(t, -4, axis=1)
        t = t + jnp.roll(t, -32, axis=1)
        t = t + jnp.roll(t, -16, axis=1)
        t = t + jnp.roll(t, -8, axis=1)
        return t[:, 0:1]

    den1 = den_of(hq1)
    den2 = den_of(hq2)

    w = 0.125 * (hq1 / (den1 + EPS) + hq2 / (den2 + EPS))

    def body(jb, num):
        kb = k_ref[0, pl.ds(jb * BK, BK), :]
        vb = v_ref[0, pl.ds(jb * BK, BK), :]

        hk, _ = _onehot_ranked(kb, tri)

        A = jax.lax.dot_general(
            q, kb, (((1,), (1,)), ((), ())),
            preferred_element_type=jnp.float32,
            precision=jax.lax.Precision.HIGHEST,
        )
        G = jax.lax.dot_general(
            w, hk, (((1,), (1,)), ((), ())),
            preferred_element_type=jnp.float32,
            precision=jax.lax.Precision.HIGHEST,
        )
        return num + jax.lax.dot_general(
            A * G, vb, (((1,), (0,)), ((), ())),
            preferred_element_type=jnp.float32,
            precision=jax.lax.Precision.HIGHEST,
        )

    num = jax.lax.fori_loop(0, s // BK, body, jnp.zeros((BQ, d), jnp.float32))
    o_ref[0] = num


def kernel(queries, keys, values):
    b, h, s, d = queries.shape
    q = queries.reshape(b * h, s, d)
    k = keys.reshape(b * h, s, d)
    v = values.reshape(b * h, s, d)
    nh = b * h

    assigns = pl.pallas_call(
        _assign_body,
        grid=(nh,),
        in_specs=[pl.BlockSpec((1, s, d), lambda hh: (hh, 0, 0))],
        out_specs=pl.BlockSpec((1, 1, s), lambda hh: (hh, 0, 0)),
        out_shape=jax.ShapeDtypeStruct((nh, 1, s), jnp.int32),
    )(k)

    zmat = pl.pallas_call(
        _z_body,
        grid=(1,),
        in_specs=[
            pl.BlockSpec((nh, 1, s), lambda _: (0, 0, 0),
                         memory_space=pltpu.SMEM),
            pl.BlockSpec((nh, s, d), lambda _: (0, 0, 0)),
        ],
        out_specs=pl.BlockSpec((nh, d, d), lambda _: (0, 0, 0)),
        out_shape=jax.ShapeDtypeStruct((nh, d, d), jnp.float32),
    )(assigns, k)

    grid = (nh, s // BQ)
    out = pl.pallas_call(
        _attn_body,
        grid=grid,
        in_specs=[
            pl.BlockSpec((1, BQ, d), lambda hh, i: (hh, i, 0)),
            pl.BlockSpec((1, s, d), lambda hh, i: (hh, 0, 0)),
            pl.BlockSpec((1, s, d), lambda hh, i: (hh, 0, 0)),
            pl.BlockSpec((1, d, d), lambda hh, i: (hh, 0, 0)),
        ],
        out_specs=pl.BlockSpec((1, BQ, d), lambda hh, i: (hh, i, 0)),
        out_shape=jax.ShapeDtypeStruct((nh, s, d), jnp.float32),
    )(q, k, v, zmat)
    return out.reshape(b, h, s, d)


# speedup vs baseline: 15.8213x; 1.0051x over previous
import jax
import jax.numpy as jnp
from jax.experimental import pallas as pl
from jax.experimental.pallas import tpu as pltpu

EPS = 1e-06
BQ = 256
BK = 512


def _tri(d):
    return (jax.lax.broadcasted_iota(jnp.int32, (d, d), 0)
            <= jax.lax.broadcasted_iota(jnp.int32, (d, d), 1)
            ).astype(jnp.float32)


def _onehot_ranked(x, tri):
    ax = jnp.abs(x)
    mx = jnp.max(ax, axis=-1, keepdims=True)
    eq = (ax == mx).astype(jnp.float32)
    cnt = jax.lax.dot_general(
        eq, tri, (((1,), (0,)), ((), ())),
        preferred_element_type=jnp.float32,
        precision=jax.lax.Precision.HIGHEST,
    )
    h1 = eq * (cnt == 1.0).astype(jnp.float32)
    h2 = eq * (cnt == 2.0).astype(jnp.float32)
    return h1, h2


def _assign_body(k_ref, a_ref):
    k = k_ref[0]
    d = k.shape[-1]
    h1, _ = _onehot_ranked(k, _tri(d))
    iota_col = jax.lax.broadcasted_iota(jnp.int32, (d, 1), 0).astype(jnp.float32)
    iota_row = iota_col.reshape(1, d)
    a = jax.lax.dot_general(
        iota_row, h1, (((1,), (1,)), ((), ())),
        preferred_element_type=jnp.float32,
        precision=jax.lax.Precision.HIGHEST,
    )
    a_ref[0] = a.astype(jnp.int32)


def _z_body(a_ref, k_ref, z_ref, *scratch):
    nh = k_ref.shape[0]
    s = k_ref.shape[1]

    for h in range(nh):
        scratch[h][...] = jnp.zeros_like(scratch[h])

    def step(i, _):
        for h in range(nh):
            c = a_ref[h, 0, i]
            scratch[h][pl.ds(c, 1), :] += k_ref[h, pl.ds(i, 1), :]
        return 0

    jax.lax.fori_loop(0, s, step, 0)
    for h in range(nh):
        z_ref[h] = scratch[h][...]


def _attn_body(q_ref, k_ref, v_ref, z_ref, o_ref):
    q = q_ref[0]
    s = k_ref.shape[1]
    d = q.shape[-1]

    tri = _tri(d)
    hq1, hq2 = _onehot_ranked(q, tri)

    z = z_ref[0]

    def den_of(hq):
        zsel = jax.lax.dot_general(
            hq, z, (((1,), (0,)), ((), ())),
            preferred_element_type=jnp.float32,
            precision=jax.lax.Precision.HIGHEST,
        )
        p = zsel * q
        t = p + jnp.roll(p, -16, axis=1)
        t = t + jnp.roll(p, -32, axis=1)
        t = t + jnp.roll(p, -48, axis=1)
        t = t + jnp.roll(t, -8, axis=1)
        t = t + jnp.roll(t, -4, axis=1)
        t = t + jnp.roll(t, -1, axis=1)
        t = t + jnp.roll(t, -2, axis=1)
        return t[:, 0:1]

    den1 = den_of(hq1)
    den2 = den_of(hq2)

    w = 0.125 * (hq1 / (den1 + EPS) + hq2 / (den2 + EPS))

    def body(jb, num):
        kb = k_ref[0, pl.ds(jb * BK, BK), :]
        vb = v_ref[0, pl.ds(jb * BK, BK), :]

        hk, _ = _onehot_ranked(kb, tri)

        A = jax.lax.dot_general(
            q, kb, (((1,), (1,)), ((), ())),
            preferred_element_type=jnp.float32,
            precision=jax.lax.Precision.HIGHEST,
        )
        G = jax.lax.dot_general(
            w, hk, (((1,), (1,)), ((), ())),
            preferred_element_type=jnp.float32,
            precision=jax.lax.Precision.HIGHEST,
        )
        return num + jax.lax.dot_general(
            A * G, vb, (((1,), (0,)), ((), ())),
            preferred_element_type=jnp.float32,
            precision=jax.lax.Precision.HIGHEST,
        )

    num = jax.lax.fori_loop(0, s // BK, body, jnp.zeros((BQ, d), jnp.float32))
    o_ref[0] = num


def kernel(queries, keys, values):
    b, h, s, d = queries.shape
    q = queries.reshape(b * h, s, d)
    k = keys.reshape(b * h, s, d)
    v = values.reshape(b * h, s, d)
    nh = b * h

    assigns = pl.pallas_call(
        _assign_body,
        grid=(nh,),
        in_specs=[pl.BlockSpec((1, s, d), lambda hh: (hh, 0, 0))],
        out_specs=pl.BlockSpec((1, 1, s), lambda hh: (hh, 0, 0)),
        out_shape=jax.ShapeDtypeStruct((nh, 1, s), jnp.int32),
    )(k)

    zmat = pl.pallas_call(
        _z_body,
        grid=(1,),
        in_specs=[
            pl.BlockSpec((nh, 1, s), lambda _: (0, 0, 0),
                         memory_space=pltpu.SMEM),
            pl.BlockSpec((nh, s, d), lambda _: (0, 0, 0)),
        ],
        out_specs=pl.BlockSpec((nh, d, d), lambda _: (0, 0, 0)),
        out_shape=jax.ShapeDtypeStruct((nh, d, d), jnp.float32),
        scratch_shapes=[pltpu.VMEM((d, d), jnp.float32)] * nh,
    )(assigns, k)

    grid = (nh, s // BQ)
    out = pl.pallas_call(
        _attn_body,
        grid=grid,
        in_specs=[
            pl.BlockSpec((1, BQ, d), lambda hh, i: (hh, i, 0)),
            pl.BlockSpec((1, s, d), lambda hh, i: (hh, 0, 0)),
            pl.BlockSpec((1, s, d), lambda hh, i: (hh, 0, 0)),
            pl.BlockSpec((1, d, d), lambda hh, i: (hh, 0, 0)),
        ],
        out_specs=pl.BlockSpec((1, BQ, d), lambda hh, i: (hh, i, 0)),
        out_shape=jax.ShapeDtypeStruct((nh, s, d), jnp.float32),
    )(q, k, v, zmat)
    return out.reshape(b, h, s, d)
